# Initial kernel scaffold; baseline (speedup 1.0000x reference)
import jax
import jax.numpy as jnp
from jax.experimental import pallas as pl


def kernel(x, edge_index, W, a0, a1):
    raise NotImplementedError("write your pallas kernel here")


# speedup vs baseline: 4.6254x; 4.6254x over previous
import jax
import jax.numpy as jnp
from jax import lax
from jax.experimental import pallas as pl
from jax.experimental.pallas import tpu as pltpu
from jax.experimental.pallas import tpu_sc as plsc

N_NODES = 10000
N_EDGES = 160000
D_IN = 256
D_OUT = 256
ALPHA = 0.2

NPAD = 10240
EPAD = 163840
NW = 32
EW = EPAD // NW
C = 64
NCH = EW // C
TW = 128
RPS = NPAD // 16
YW = 384
BIG = 3.0e38

_MESH = plsc.VectorSubcoreMesh(core_axis_name="c", subcore_axis_name="s")
_SC_PARAMS = pltpu.CompilerParams(needs_layout_passes=False)


def _k1_body(x_ref, w_ref, y_ref):
    y_ref[...] = jnp.dot(x_ref[...], w_ref[...],
                         preferred_element_type=jnp.float32)


def _k1(xpad, wbig):
    blk = 1024
    return pl.pallas_call(
        _k1_body,
        grid=(NPAD // blk,),
        in_specs=[
            pl.BlockSpec((blk, D_IN), lambda i: (i, 0)),
            pl.BlockSpec((D_IN, YW), lambda i: (0, 0)),
        ],
        out_specs=pl.BlockSpec((blk, YW), lambda i: (i, 0)),
        out_shape=jax.ShapeDtypeStruct((NPAD, YW), jnp.float32),
    )(xpad, wbig)


def _k2_body(ar_h, ac_h, row_h, col_h, s_h, mm_h,
             art, act, rowv, colv, sv, mnv, mxv):
    cid = lax.axis_index("c")
    sid = lax.axis_index("s")
    w = sid * 2 + cid
    pltpu.sync_copy(ar_h, art)
    pltpu.sync_copy(ac_h, act)
    pltpu.sync_copy(row_h.at[w], rowv)
    pltpu.sync_copy(col_h.at[w], colv)
    mn = jnp.full((16,), BIG, jnp.float32)
    mx = jnp.full((16,), -BIG, jnp.float32)
    lane = lax.iota(jnp.int32, 16)
    ebase = w * EW
    for g in range(NCH):
        for j in range(C // 16):
            rv = rowv[g, pl.ds(j * 16, 16)]
            cv = colv[g, pl.ds(j * 16, 16)]
            s = plsc.load_gather(art, [rv]) + plsc.load_gather(act, [cv])
            s = jnp.maximum(s, ALPHA * s)
            sv[pl.ds(g * C + j * 16, 16)] = s
            valid = (ebase + g * C + j * 16 + lane) < N_EDGES
            mn = jnp.minimum(mn, jnp.where(valid, s, BIG))
            mx = jnp.maximum(mx, jnp.where(valid, s, -BIG))
    mnv[...] = mn
    mxv[...] = mx
    pltpu.sync_copy(sv, s_h.at[w])
    pltpu.sync_copy(mnv, mm_h.at[0, w])
    pltpu.sync_copy(mxv, mm_h.at[1, w])


_k2 = pl.kernel(
    _k2_body,
    out_type=(
        jax.ShapeDtypeStruct((NW, EW), jnp.float32),
        jax.ShapeDtypeStruct((2, NW, 16), jnp.float32),
    ),
    mesh=_MESH,
    scratch_types=[
        pltpu.VMEM((NPAD,), jnp.float32),
        pltpu.VMEM((NPAD,), jnp.float32),
        pltpu.VMEM((NCH, C), jnp.int32),
        pltpu.VMEM((NCH, C), jnp.int32),
        pltpu.VMEM((EW,), jnp.float32),
        pltpu.VMEM((16,), jnp.float32),
        pltpu.VMEM((16,), jnp.float32),
    ],
    compiler_params=_SC_PARAMS,
)


def _k3_body(s_h, mm_h, row_h, col_h, tab0_h, tab1_h, hp_h,
             rowv, colv, attv, gbuf, mmv, sem, acc):
    cid = lax.axis_index("c")
    sid = lax.axis_index("s")
    w = sid * 2 + cid
    pltpu.sync_copy(mm_h, mmv)
    pltpu.sync_copy(row_h.at[w], rowv)
    pltpu.sync_copy(col_h.at[w], colv)
    pltpu.sync_copy(s_h.at[w], attv)
    mn = jnp.full((16,), BIG, jnp.float32)
    mx = jnp.full((16,), -BIG, jnp.float32)
    for i in range(NW):
        mn = jnp.minimum(mn, mmv[0, i, pl.ds(0, 16)])
        mx = jnp.maximum(mx, mmv[1, i, pl.ds(0, 16)])
    gmin = jnp.min(mn)
    inv = 1.0 / jnp.full((16,), jnp.max(mx) - gmin, jnp.float32)
    zeros16 = jnp.zeros((16,), jnp.float32)
    lane = lax.iota(jnp.int32, 16)
    ebase = w * EW

    def _expbody(t, carry):
        s = attv[pl.ds(t * 16, 16)]
        a = jnp.exp((s - gmin) * inv)
        valid = (ebase + t * 16 + lane) < N_EDGES
        attv[pl.ds(t * 16, 16)] = jnp.where(valid, a, 0.0)
        return carry

    lax.fori_loop(0, EW // 16, _expbody, 0)

    for phase, tab in enumerate((tab0_h, tab1_h)):
        def _zb(k, carry):
            for l in range(16):
                for j in range(TW // 16):
                    gbuf[k * 16 + l, pl.ds(j * 16, 16)] = zeros16
            return carry

        lax.fori_loop(0, C // 16, _zb, 0)
        for t in range(RPS // C):
            pltpu.sync_copy(gbuf, acc.at[pl.ds(sid * RPS + t * C, C)])
        plsc.subcore_barrier()

        def _chunk(g, carry):
            pltpu.async_copy(tab.at[colv.at[g]], gbuf, sem).wait()

            def _scale(k, carry2):
                at = attv[pl.ds(g * C + k * 16, 16)]
                for l in range(16):
                    a = at[l]
                    for j in range(TW // 16):
                        gbuf[k * 16 + l, pl.ds(j * 16, 16)] = (
                            gbuf[k * 16 + l, pl.ds(j * 16, 16)] * a
                        )
                return carry2

            lax.fori_loop(0, C // 16, _scale, 0)
            pltpu.sync_copy(gbuf, acc.at[rowv.at[g]], add=True)
            return carry

        lax.fori_loop(0, NCH, _chunk, 0)
        plsc.subcore_barrier()
        pltpu.sync_copy(
            acc.at[pl.ds(sid * RPS, RPS)],
            hp_h.at[cid, phase, pl.ds(sid * RPS, RPS)],
        )
        plsc.subcore_barrier()


_k3 = pl.kernel(
    _k3_body,
    out_type=jax.ShapeDtypeStruct((2, 2, NPAD, TW), jnp.float32),
    mesh=_MESH,
    scratch_types=[
        pltpu.VMEM((NCH, C), jnp.int32),
        pltpu.VMEM((NCH, C), jnp.int32),
        pltpu.VMEM((EW,), jnp.float32),
        pltpu.VMEM((C, TW), jnp.float32),
        pltpu.VMEM((2, NW, 16), jnp.float32),
        pltpu.SemaphoreType.DMA,
        pltpu.VMEM_SHARED((NPAD, TW), jnp.float32),
    ],
    compiler_params=_SC_PARAMS,
)


def _k3b_body(s_h, mm_h, rowf_h, rsa_h, rs_h,
              rowfv, attv, rsl, rsm, rso, mmv):
    cid = lax.axis_index("c")
    sid = lax.axis_index("s")
    w = sid * 2 + cid
    pltpu.sync_copy(mm_h, mmv)
    pltpu.sync_copy(rowf_h.at[w], rowfv)
    pltpu.sync_copy(s_h.at[w], attv)
    mn = jnp.full((16,), BIG, jnp.float32)
    mx = jnp.full((16,), -BIG, jnp.float32)
    for i in range(NW):
        mn = jnp.minimum(mn, mmv[0, i, pl.ds(0, 16)])
        mx = jnp.maximum(mx, mmv[1, i, pl.ds(0, 16)])
    gmin = jnp.min(mn)
    inv = 1.0 / jnp.full((16,), jnp.max(mx) - gmin, jnp.float32)
    zeros16 = jnp.zeros((16,), jnp.float32)
    lane = lax.iota(jnp.int32, 16)
    ebase = w * EW

    def _zrs(t, carry):
        rsl[pl.ds(t * 16, 16)] = zeros16
        return carry

    lax.fori_loop(0, NPAD // 16, _zrs, 0)

    def _rsbody(t, carry):
        s = attv[pl.ds(t * 16, 16)]
        a = jnp.exp((s - gmin) * inv)
        valid = (ebase + t * 16 + lane) < N_EDGES
        a = jnp.where(valid, a, 0.0)
        rv = rowfv[pl.ds(t * 16, 16)]
        plsc.addupdate_scatter(rsl, [rv], a)
        return carry

    lax.fori_loop(0, EW // 16, _rsbody, 0)

    pltpu.sync_copy(rsl, rsa_h.at[cid, sid])
    plsc.subcore_barrier()
    for p in range(RPS // 128):
        pltpu.sync_copy(
            rsa_h.at[cid, pl.ds(0, 16), pl.ds(sid * RPS + p * 128, 128)], rsm)

        def _merge(u, carry):
            tot = rsm[0, pl.ds(u * 16, 16)]
            for k in range(1, 16):
                tot = tot + rsm[k, pl.ds(u * 16, 16)]
            rso[pl.ds(p * 128 + u * 16, 16)] = tot
            return carry

        lax.fori_loop(0, 8, _merge, 0)
    pltpu.sync_copy(rso, rs_h.at[cid, pl.ds(sid * RPS, RPS)])


_k3b = pl.kernel(
    _k3b_body,
    out_type=(
        jax.ShapeDtypeStruct((2, 16, NPAD), jnp.float32),
        jax.ShapeDtypeStruct((2, NPAD), jnp.float32),
    ),
    mesh=_MESH,
    scratch_types=[
        pltpu.VMEM((EW,), jnp.int32),
        pltpu.VMEM((EW,), jnp.float32),
        pltpu.VMEM((NPAD,), jnp.float32),
        pltpu.VMEM((16, 128), jnp.float32),
        pltpu.VMEM((RPS,), jnp.float32),
        pltpu.VMEM((2, NW, 16), jnp.float32),
    ],
    compiler_params=_SC_PARAMS,
)


def _k4_body(hp_ref, rs_ref, o_ref):
    h0 = hp_ref[0, 0] + hp_ref[1, 0]
    h1 = hp_ref[0, 1] + hp_ref[1, 1]
    rs = (rs_ref[0, :] + rs_ref[1, :])[:, None]
    o_ref[...] = jnp.concatenate([h0 / rs, h1 / rs], axis=1)


def _k4(hp, rs):
    blk = 512
    return pl.pallas_call(
        _k4_body,
        grid=(NPAD // blk,),
        in_specs=[
            pl.BlockSpec((2, 2, blk, TW), lambda i: (0, 0, i, 0)),
            pl.BlockSpec((2, blk), lambda i: (0, i)),
        ],
        out_specs=pl.BlockSpec((blk, D_OUT), lambda i: (i, 0)),
        out_shape=jax.ShapeDtypeStruct((NPAD, D_OUT), jnp.float32),
    )(hp, rs)


@jax.jit
def kernel(x, edge_index, W, a0, a1):
    f32, i32 = jnp.float32, jnp.int32
    xpad = jnp.zeros((NPAD, D_IN), f32).at[:N_NODES].set(x)
    wa0 = jnp.dot(W, a0[0])
    wa1 = jnp.dot(W, a1[0])
    wbig = (
        jnp.zeros((D_IN, YW), f32)
        .at[:, 0:D_OUT].set(W)
        .at[:, 256].set(wa0)
        .at[:, 257].set(wa1)
    )
    y = _k1(xpad, wbig)
    tab0 = lax.slice(y, (0, 0), (NPAD, TW))
    tab1 = lax.slice(y, (0, TW), (NPAD, 2 * TW))
    ar = y[:, 256]
    ac = y[:, 257]
    rowp = jnp.full((EPAD,), NPAD - 1, i32).at[:N_EDGES].set(edge_index[0])
    colp = jnp.zeros((EPAD,), i32).at[:N_EDGES].set(edge_index[1])
    row2 = rowp.reshape(NW, NCH, C)
    rowf = rowp.reshape(NW, EW)
    col2 = colp.reshape(NW, NCH, C)
    s_h, mm_h = _k2(ar, ac, row2, col2)
    hp = _k3(s_h, mm_h, row2, col2, tab0, tab1)
    _, rs = _k3b(s_h, mm_h, rowf)
    out = _k4(hp, rs)
    return out[:N_NODES]


# speedup vs baseline: 5.3940x; 1.1662x over previous
import jax
import jax.numpy as jnp
from jax import lax
from jax.experimental import pallas as pl
from jax.experimental.pallas import tpu as pltpu
from jax.experimental.pallas import tpu_sc as plsc

N_NODES = 10000
N_EDGES = 160000
D_IN = 256
D_OUT = 256
ALPHA = 0.2

NPAD = 10112
NRS = 10240
EPAD = 163840
NW = 32
EW = EPAD // NW
C = 64
NCH = EW // C
TW = 128
RPS = NPAD // 16
RPSR = NRS // 16
YW = 384
BIG = 3.0e38

_MESH = plsc.VectorSubcoreMesh(core_axis_name="c", subcore_axis_name="s")
_SC_PARAMS = pltpu.CompilerParams(needs_layout_passes=False, internal_scratch_in_bytes=16384)


def _k1_body(x_ref, w_ref, y_ref):
    y_ref[...] = jnp.dot(x_ref[...], w_ref[...],
                         preferred_element_type=jnp.float32)


def _k1(xpad, wbig):
    blk = 632
    return pl.pallas_call(
        _k1_body,
        grid=(NPAD // blk,),
        in_specs=[
            pl.BlockSpec((blk, D_IN), lambda i: (i, 0)),
            pl.BlockSpec((D_IN, YW), lambda i: (0, 0)),
        ],
        out_specs=pl.BlockSpec((blk, YW), lambda i: (i, 0)),
        out_shape=jax.ShapeDtypeStruct((NPAD, YW), jnp.float32),
    )(xpad, wbig)


def _k2_body(ar_h, ac_h, row_h, col_h, s_h, mm_h,
             art, act, rowv, colv, sv, mnv, mxv):
    cid = lax.axis_index("c")
    sid = lax.axis_index("s")
    w = sid * 2 + cid
    pltpu.sync_copy(ar_h, art)
    pltpu.sync_copy(ac_h, act)
    pltpu.sync_copy(row_h.at[w], rowv)
    pltpu.sync_copy(col_h.at[w], colv)
    mn = jnp.full((16,), BIG, jnp.float32)
    mx = jnp.full((16,), -BIG, jnp.float32)
    lane = lax.iota(jnp.int32, 16)
    ebase = w * EW
    for g in range(NCH):
        for j in range(C // 16):
            rv = rowv[g, pl.ds(j * 16, 16)]
            cv = colv[g, pl.ds(j * 16, 16)]
            s = plsc.load_gather(art, [rv]) + plsc.load_gather(act, [cv])
            s = jnp.maximum(s, ALPHA * s)
            sv[pl.ds(g * C + j * 16, 16)] = s
            valid = (ebase + g * C + j * 16 + lane) < N_EDGES
            mn = jnp.minimum(mn, jnp.where(valid, s, BIG))
            mx = jnp.maximum(mx, jnp.where(valid, s, -BIG))
    mnv[...] = mn
    mxv[...] = mx
    pltpu.sync_copy(sv, s_h.at[w])
    pltpu.sync_copy(mnv, mm_h.at[0, w])
    pltpu.sync_copy(mxv, mm_h.at[1, w])


_k2 = pl.kernel(
    _k2_body,
    out_type=(
        jax.ShapeDtypeStruct((NW, EW), jnp.float32),
        jax.ShapeDtypeStruct((2, NW, 16), jnp.float32),
    ),
    mesh=_MESH,
    scratch_types=[
        pltpu.VMEM((NPAD,), jnp.float32),
        pltpu.VMEM((NPAD,), jnp.float32),
        pltpu.VMEM((NCH, C), jnp.int32),
        pltpu.VMEM((NCH, C), jnp.int32),
        pltpu.VMEM((EW,), jnp.float32),
        pltpu.VMEM((16,), jnp.float32),
        pltpu.VMEM((16,), jnp.float32),
    ],
    compiler_params=_SC_PARAMS,
)


def _k3p_body(att_h, row_h, col_h, tab_h, hp_h,
              rowv, colv, attv, gbuf0, gbuf1, idb0, idb1, rwb,
              sem0, sem1, acc):
    cid = lax.axis_index("c")
    sid = lax.axis_index("s")
    w = sid * 2 + cid
    pltpu.sync_copy(row_h.at[w], rowv)
    pltpu.sync_copy(col_h.at[w], colv)
    pltpu.sync_copy(att_h.at[w], attv)
    zeros16 = jnp.zeros((16,), jnp.float32)

    def _fill(dst, srcref, g):
        for j in range(C // 16):
            dst[pl.ds(j * 16, 16)] = srcref[g, pl.ds(j * 16, 16)]

    def _scale_scatter(buf, g):
        def _scale(k, carry2):
            at = attv[pl.ds(g * C + k * 16, 16)]
            for l in range(16):
                a = at[l]
                for j in range(TW // 16):
                    buf[k * 16 + l, pl.ds(j * 16, 16)] = (
                        buf[k * 16 + l, pl.ds(j * 16, 16)] * a
                    )
            return carry2

        lax.fori_loop(0, C // 16, _scale, 0)
        _fill(rwb, rowv, g)
        pltpu.sync_copy(buf, acc.at[rwb], add=True)

    def _zb(k, carry):
        for l in range(16):
            for j in range(TW // 16):
                gbuf0[k * 16 + l, pl.ds(j * 16, 16)] = zeros16
        return carry

    lax.fori_loop(0, C // 16, _zb, 0)
    for t in range(RPS // C):
        pltpu.sync_copy(gbuf0, acc.at[pl.ds(sid * RPS + t * C, C)])
    if RPS % C:
        pltpu.sync_copy(
            gbuf0.at[pl.ds(0, RPS % C)],
            acc.at[pl.ds(sid * RPS + (RPS // C) * C, RPS % C)])
    plsc.subcore_barrier()

    _fill(idb0, colv, 0)
    pltpu.async_copy(tab_h.at[idb0], gbuf0, sem0)

    def _pair(h, carry):
        g = h * 2
        _fill(idb1, colv, g + 1)
        pltpu.async_copy(tab_h.at[idb1], gbuf1, sem1)
        pltpu.make_async_copy(tab_h.at[pl.ds(0, C)], gbuf0, sem0).wait()
        _scale_scatter(gbuf0, g)

        @pl.when(h < NCH // 2 - 1)
        def _fire_next():
            _fill(idb0, colv, g + 2)
            pltpu.async_copy(tab_h.at[idb0], gbuf0, sem0)

        pltpu.make_async_copy(tab_h.at[pl.ds(0, C)], gbuf1, sem1).wait()
        _scale_scatter(gbuf1, g + 1)
        return carry

    lax.fori_loop(0, NCH // 2, _pair, 0)
    plsc.subcore_barrier()
    pltpu.sync_copy(
        acc.at[pl.ds(sid * RPS, RPS)],
        hp_h.at[cid, pl.ds(sid * RPS, RPS)],
    )


_k3p = pl.kernel(
    _k3p_body,
    out_type=jax.ShapeDtypeStruct((2, NPAD, TW), jnp.float32),
    mesh=_MESH,
    scratch_types=[
        pltpu.VMEM((NCH, C), jnp.int32),
        pltpu.VMEM((NCH, C), jnp.int32),
        pltpu.VMEM((EW,), jnp.float32),
        pltpu.VMEM((C, TW), jnp.float32),
        pltpu.VMEM((C, TW), jnp.float32),
        pltpu.VMEM((C,), jnp.int32),
        pltpu.VMEM((C,), jnp.int32),
        pltpu.VMEM((C,), jnp.int32),
        pltpu.SemaphoreType.DMA,
        pltpu.SemaphoreType.DMA,
        pltpu.VMEM_SHARED((NPAD, TW), jnp.float32),
    ],
    compiler_params=_SC_PARAMS,
)


def _k3b_body(s_h, mm_h, rowf_h, rsa_h, rs_h, att_h,
              rowfv, attv, rsl, rsm, rso, mmv):
    cid = lax.axis_index("c")
    sid = lax.axis_index("s")
    w = sid * 2 + cid
    pltpu.sync_copy(mm_h, mmv)
    pltpu.sync_copy(rowf_h.at[w], rowfv)
    pltpu.sync_copy(s_h.at[w], attv)
    mn = jnp.full((16,), BIG, jnp.float32)
    mx = jnp.full((16,), -BIG, jnp.float32)
    for i in range(NW):
        mn = jnp.minimum(mn, mmv[0, i, pl.ds(0, 16)])
        mx = jnp.maximum(mx, mmv[1, i, pl.ds(0, 16)])
    gmin = jnp.min(mn)
    inv = 1.0 / jnp.full((16,), jnp.max(mx) - gmin, jnp.float32)
    zeros16 = jnp.zeros((16,), jnp.float32)
    lane = lax.iota(jnp.int32, 16)
    ebase = w * EW

    def _zrs(t, carry):
        rsl[pl.ds(t * 16, 16)] = zeros16
        return carry

    lax.fori_loop(0, NRS // 16, _zrs, 0)

    def _rsbody(t, carry):
        s = attv[pl.ds(t * 16, 16)]
        a = jnp.exp((s - gmin) * inv)
        valid = (ebase + t * 16 + lane) < N_EDGES
        a = jnp.where(valid, a, 0.0)
        attv[pl.ds(t * 16, 16)] = a
        rv = rowfv[pl.ds(t * 16, 16)]
        plsc.addupdate_scatter(rsl, [rv], a)
        return carry

    lax.fori_loop(0, EW // 16, _rsbody, 0)
    pltpu.sync_copy(attv, att_h.at[w])

    pltpu.sync_copy(rsl, rsa_h.at[cid, sid])
    plsc.subcore_barrier()
    for p in range(RPSR // 128):
        pltpu.sync_copy(
            rsa_h.at[cid, pl.ds(0, 16), pl.ds(sid * RPSR + p * 128, 128)], rsm)

        def _merge(u, carry):
            tot = rsm[0, pl.ds(u * 16, 16)]
            for k in range(1, 16):
                tot = tot + rsm[k, pl.ds(u * 16, 16)]
            rso[pl.ds(p * 128 + u * 16, 16)] = tot
            return carry

        lax.fori_loop(0, 8, _merge, 0)
    pltpu.sync_copy(rso, rs_h.at[cid, pl.ds(sid * RPSR, RPSR)])


_k3b = pl.kernel(
    _k3b_body,
    out_type=(
        jax.ShapeDtypeStruct((2, 16, NRS), jnp.float32),
        jax.ShapeDtypeStruct((2, NRS), jnp.float32),
        jax.ShapeDtypeStruct((NW, EW), jnp.float32),
    ),
    mesh=_MESH,
    scratch_types=[
        pltpu.VMEM((EW,), jnp.int32),
        pltpu.VMEM((EW,), jnp.float32),
        pltpu.VMEM((NRS,), jnp.float32),
        pltpu.VMEM((16, 128), jnp.float32),
        pltpu.VMEM((RPSR,), jnp.float32),
        pltpu.VMEM((2, NW, 16), jnp.float32),
    ],
    compiler_params=_SC_PARAMS,
)


def _k4_body(hp0_ref, hp1_ref, rs_ref, o_ref):
    h0 = hp0_ref[0] + hp0_ref[1]
    h1 = hp1_ref[0] + hp1_ref[1]
    rs = rs_ref[0] + rs_ref[1]
    o_ref[...] = jnp.concatenate([h0 / rs, h1 / rs], axis=1)


def _k4(hp0, hp1, rs):
    blk = 632
    return pl.pallas_call(
        _k4_body,
        grid=(NPAD // blk,),
        in_specs=[
            pl.BlockSpec((2, blk, TW), lambda i: (0, i, 0)),
            pl.BlockSpec((2, blk, TW), lambda i: (0, i, 0)),
            pl.BlockSpec((2, blk, 1), lambda i: (0, i, 0)),
        ],
        out_specs=pl.BlockSpec((blk, D_OUT), lambda i: (i, 0)),
        out_shape=jax.ShapeDtypeStruct((NPAD, D_OUT), jnp.float32),
    )(hp0, hp1, rs)


@jax.jit
def kernel(x, edge_index, W, a0, a1):
    f32, i32 = jnp.float32, jnp.int32
    xpad = jnp.zeros((NPAD, D_IN), f32).at[:N_NODES].set(x)
    wa0 = jnp.dot(W, a0[0])
    wa1 = jnp.dot(W, a1[0])
    wbig = (
        jnp.zeros((D_IN, YW), f32)
        .at[:, 0:D_OUT].set(W)
        .at[:, 256].set(wa0)
        .at[:, 257].set(wa1)
    )
    y = _k1(xpad, wbig)
    tab0 = lax.slice(y, (0, 0), (NPAD, TW))
    tab1 = lax.slice(y, (0, TW), (NPAD, 2 * TW))
    ar = y[:, 256]
    ac = y[:, 257]
    rowp = jnp.full((EPAD,), NPAD - 1, i32).at[:N_EDGES].set(edge_index[0])
    colp = jnp.zeros((EPAD,), i32).at[:N_EDGES].set(edge_index[1])
    row2 = rowp.reshape(NW, NCH, C)
    rowf = rowp.reshape(NW, EW)
    col2 = colp.reshape(NW, NCH, C)
    s_h, mm_h = _k2(ar, ac, row2, col2)
    _, rs, att_h = _k3b(s_h, mm_h, rowf)
    hp0 = _k3p(att_h, row2, col2, tab0)
    hp1 = _k3p(att_h, row2, col2, tab1)
    out = _k4(hp0, hp1, rs[:, :NPAD, None])
    return out[:N_NODES]
